# baseline (speedup 1.0000x reference)
import functools

import jax
import jax.numpy as jnp
import numpy as np
from jax import lax
from jax.experimental import pallas as pl
from jax.experimental.pallas import tpu as pltpu
from jax.experimental.pallas import tpu_sc as plsc

F = 128
K = 32
H = 4
DH = F // H
CUTOFF = 5.0
N = 10000
NPAD = 10240
E = 320000
EPAD = 327680
BN = 1024
GN = NPAD // BN
BE = 2048
GE = EPAD // BE
NC = 2
NS = 16
NW = NC * NS
CH = 128
EW = EPAD // NW
NCHUNK = EW // CH
ROWS_N = NPAD // NS

@functools.lru_cache(maxsize=1)
def _sc_mesh():
    return plsc.VectorSubcoreMesh(
        core_axis_name="c", subcore_axis_name="s",
        num_cores=NC, num_subcores=NS)

_f32 = jnp.float32


def _silu(x):
    return x / (1.0 + jnp.exp(-x))


def _node_proj_body(z_ref, emb_ref, wq_ref, wkv_ref,
                    h_ref, q_ref, kv_ref):
    z = z_ref[...]
    col = lax.broadcasted_iota(jnp.int32, (BN, 128), 1)
    oh = (col == z).astype(_f32)
    h = jnp.dot(oh, emb_ref[...], preferred_element_type=_f32)
    h_ref[...] = h
    q_ref[...] = jnp.dot(h, wq_ref[...], preferred_element_type=_f32)
    kv_ref[...] = jnp.dot(h, wkv_ref[...], preferred_element_type=_f32)


def _node_proj(z2, embp, wq, wkv):
    full = lambda a: pl.BlockSpec(a.shape, lambda i: (0,) * a.ndim)
    return pl.pallas_call(
        _node_proj_body,
        grid=(GN,),
        in_specs=[pl.BlockSpec((BN, 1), lambda i: (i, 0)),
                  full(embp), full(wq), full(wkv)],
        out_specs=[pl.BlockSpec((BN, F), lambda i: (i, 0)),
                   pl.BlockSpec((BN, F), lambda i: (i, 0)),
                   pl.BlockSpec((BN, 2 * F), lambda i: (i, 0))],
        out_shape=[jax.ShapeDtypeStruct((NPAD, F), _f32),
                   jax.ShapeDtypeStruct((NPAD, F), _f32),
                   jax.ShapeDtypeStruct((NPAD, 2 * F), _f32)],
    )(z2, embp, wq, wkv)


def _edge_geom(dr):
    r2 = jnp.sum(dr * dr, axis=1, keepdims=True)
    r = jnp.sqrt(r2 + 1e-12)
    rc = jnp.minimum(r, CUTOFF)
    cut = 0.5 * (jnp.cos(jnp.pi * rc / CUTOFF) + 1.0)
    cut = cut * (r < CUTOFF).astype(_f32)
    centers = (lax.broadcasted_iota(jnp.int32, (1, K), 1).astype(_f32)
               * (CUTOFF / (K - 1)))
    width = CUTOFF / K
    rbf = jnp.exp(-0.5 * ((r - centers) / width) ** 2)
    return r, cut, rbf


def _edge_feat_body(dr_ref, qg_ref, kvg_ref,
                    fw1_ref, fb1_ref, fw2_ref, fb2_ref, msg_ref):
    dr = dr_ref[...]
    _, cut, rbf = _edge_geom(dr)
    s1 = _silu(jnp.dot(rbf, fw1_ref[...], preferred_element_type=_f32)
               + fb1_ref[...])
    wf = jnp.dot(s1, fw2_ref[...], preferred_element_type=_f32) + fb2_ref[...]
    kvg = kvg_ref[...]
    t = qg_ref[...] * kvg[:, 0:F] * wf * (cut * (1.0 / np.sqrt(DH)))
    hr = lax.broadcasted_iota(jnp.int32, (F, F), 0) // DH
    hc = lax.broadcasted_iota(jnp.int32, (F, F), 1) // DH
    s_mat = (hr == hc).astype(_f32)
    alpha = jnp.dot(t, s_mat, preferred_element_type=_f32)
    msg_ref[...] = alpha * kvg[:, F:2 * F]


def _edge_feat(drp, qg, kvg, fw1, fb1, fw2, fb2):
    full = lambda a: pl.BlockSpec(a.shape, lambda i: (0,) * a.ndim)
    eb = pl.BlockSpec((BE, F), lambda i: (i, 0))
    return pl.pallas_call(
        _edge_feat_body,
        grid=(GE,),
        in_specs=[pl.BlockSpec((BE, 3), lambda i: (i, 0)), eb,
                  pl.BlockSpec((BE, 2 * F), lambda i: (i, 0)),
                  full(fw1), full(fb1), full(fw2), full(fb2)],
        out_specs=eb,
        out_shape=jax.ShapeDtypeStruct((EPAD, F), _f32),
    )(drp, qg, kvg, fw1, fb1, fw2, fb2)


def _node_upd_body(h_ref, a0_ref, a1_ref, wo_ref, wg_ref, h2_ref, g16_ref):
    agg = a0_ref[...] + a1_ref[...]
    h2 = h_ref[...] + jnp.dot(agg, wo_ref[...], preferred_element_type=_f32)
    h2_ref[...] = h2
    g16_ref[...] = jnp.dot(h2, wg_ref[...], preferred_element_type=_f32)


def _node_upd(h, aggf, wo, wg16):
    full = lambda a: pl.BlockSpec(a.shape, lambda i: (0,) * a.ndim)
    nb = pl.BlockSpec((BN, F), lambda i: (i, 0))
    return pl.pallas_call(
        _node_upd_body,
        grid=(GN,),
        in_specs=[nb, nb, pl.BlockSpec((BN, F), lambda i: (i + GN, 0)),
                  full(wo), full(wg16)],
        out_specs=[nb, nb],
        out_shape=[jax.ShapeDtypeStruct((NPAD, F), _f32),
                   jax.ShapeDtypeStruct((NPAD, F), _f32)],
    )(h, aggf, aggf, wo, wg16)


def _edge_geo_body(dr_ref, gi_ref, gj_ref,
                   gw1_ref, gb1_ref, gw2_ref, gb2_ref, ctr_ref):
    dr = dr_ref[...]
    r, cut, rbf = _edge_geom(dr)
    u = dr / r
    s1 = _silu(jnp.dot(rbf, gw1_ref[...], preferred_element_type=_f32)
               + gb1_ref[...])
    wg = jnp.dot(s1, gw2_ref[...], preferred_element_type=_f32) + gb2_ref[...]
    gi = gi_ref[...]
    gj = gj_ref[...]
    ag = gi[:, 0:3] * wg * gj[:, 4:7] * cut
    c16 = lax.broadcasted_iota(jnp.int32, (1, F), 1)
    agm = (ag[:, 0:1] * (c16 < 3).astype(_f32)
           + ag[:, 1:2] * ((c16 >= 3) & (c16 < 8)).astype(_f32)
           + ag[:, 2:3] * ((c16 >= 8) & (c16 < 15)).astype(_f32))
    x, y, z = u[:, 0:1], u[:, 1:2], u[:, 2:3]
    xx, yy, zz = x * x, y * y, z * z
    terms = [x, y, z,
             x * y, y * z, 3.0 * zz - 1.0, x * z, xx - yy,
             y * (3.0 * xx - yy), x * y * z, y * (5.0 * zz - 1.0),
             z * (5.0 * zz - 3.0), x * (5.0 * zz - 1.0),
             z * (xx - yy), x * (xx - 3.0 * yy)]
    sph = terms[0] * (c16 == 0).astype(_f32)
    for ti in range(1, 15):
        sph = sph + terms[ti] * (c16 == ti).astype(_f32)
    ctr_ref[...] = agm * sph


def _edge_geo(drp, gi, gj, gw1, gb1, gw2, gb2):
    full = lambda a: pl.BlockSpec(a.shape, lambda i: (0,) * a.ndim)
    eb = pl.BlockSpec((BE, F), lambda i: (i, 0))
    return pl.pallas_call(
        _edge_geo_body,
        grid=(GE,),
        in_specs=[pl.BlockSpec((BE, 3), lambda i: (i, 0)), eb, eb,
                  full(gw1), full(gb1), full(gw2), full(gb2)],
        out_specs=eb,
        out_shape=jax.ShapeDtypeStruct((EPAD, F), _f32),
    )(drp, gi, gj, gw1, gb1, gw2, gb2)


def _node_final_body(h2_ref, c0_ref, c1_ref, z_ref, w1h_ref, w1d_ref, b1_ref,
                     w2h_ref, b2h_ref, row1_ref, rob1_ref, row2_ref, rob2_ref,
                     ss_ref, out_ref):
    chi = c0_ref[...] + c1_ref[...]
    d1 = jnp.sum(chi[:, 0:3] ** 2, axis=1, keepdims=True)
    d2 = jnp.sum(chi[:, 3:8] ** 2, axis=1, keepdims=True)
    d3 = jnp.sum(chi[:, 8:15] ** 2, axis=1, keepdims=True)
    h2 = h2_ref[...]
    w1d = w1d_ref[...]
    pre = (jnp.dot(h2, w1h_ref[...], preferred_element_type=_f32)
           + d1 * w1d[0:1, :] + d2 * w1d[1:2, :] + d3 * w1d[2:3, :]
           + b1_ref[...])
    s = _silu(pre)
    h3 = h2 + jnp.dot(s, w2h_ref[...], preferred_element_type=_f32) + b2h_ref[...]
    t = _silu(jnp.dot(h3, row1_ref[...], preferred_element_type=_f32)
              + rob1_ref[...])
    e = jnp.dot(t, row2_ref[...], preferred_element_type=_f32) + rob2_ref[...]
    z = z_ref[...]
    nm = (z > 0).astype(_f32)
    oh = (lax.broadcasted_iota(jnp.int32, (BN, 128), 1) == z).astype(_f32)
    ss = jnp.dot(oh, ss_ref[...], preferred_element_type=_f32)
    out_ref[...] = ss[:, 0:1] * (e * nm) + ss[:, 1:2]


def _node_final(h2, chif, z2, w1h, w1d, b1, w2h, b2h,
                row1, rob1, row2, rob2, sspack):
    full = lambda a: pl.BlockSpec(a.shape, lambda i: (0,) * a.ndim)
    nb = pl.BlockSpec((BN, F), lambda i: (i, 0))
    return pl.pallas_call(
        _node_final_body,
        grid=(GN,),
        in_specs=[nb,
                  nb, pl.BlockSpec((BN, F), lambda i: (i + GN, 0)),
                  pl.BlockSpec((BN, 1), lambda i: (i, 0)),
                  full(w1h), full(w1d), full(b1), full(w2h), full(b2h),
                  full(row1), full(rob1), full(row2), full(rob2),
                  full(sspack)],
        out_specs=pl.BlockSpec((BN, 1), lambda i: (i, 0)),
        out_shape=jax.ShapeDtypeStruct((NPAD, 1), _f32),
    )(h2, chif, chif, z2, w1h, w1d, b1, w2h, b2h,
      row1, rob1, row2, rob2, sspack)


def _sc_gather(tables, idx2ds, widths):
    nt = len(tables)
    scratch = ([pltpu.VMEM((NCHUNK, CH), jnp.int32) for _ in range(nt)]
               + [pltpu.VMEM((CH, w), _f32) for w in widths for _ in (0, 1)]
               + [pltpu.SemaphoreType.DMA] * (4 * nt))

    @functools.partial(
        pl.kernel,
        out_type=[jax.ShapeDtypeStruct((EPAD, w), _f32) for w in widths],
        mesh=_sc_mesh(),
        scratch_types=scratch,
    )
    def body(*refs):
        tabs = refs[:nt]
        idxh = refs[nt:2 * nt]
        outs = refs[2 * nt:3 * nt]
        idxv = refs[3 * nt:4 * nt]
        bufs = [refs[4 * nt + 2 * t:4 * nt + 2 * t + 2] for t in range(nt)]
        semg = [refs[6 * nt + 2 * t:6 * nt + 2 * t + 2] for t in range(nt)]
        semo = [refs[8 * nt + 2 * t:8 * nt + 2 * t + 2] for t in range(nt)]
        w = lax.axis_index("s") * NC + lax.axis_index("c")
        ebase = w * EW
        rbase = w * NCHUNK
        for t in range(nt):
            pltpu.sync_copy(idxh[t].at[pl.ds(rbase, NCHUNK)], idxv[t])

        def g(t, b, ch):
            pltpu.async_copy(tabs[t].at[idxv[t].at[ch]], bufs[t][b], semg[t][b])

        def g_drain(t, b):
            pltpu.make_async_copy(
                tabs[t].at[pl.ds(0, CH)], bufs[t][b], semg[t][b]).wait()

        def co(t, b, ch):
            pltpu.async_copy(bufs[t][b],
                             outs[t].at[pl.ds(ebase + ch * CH, CH)],
                             semo[t][b])

        def co_drain(t, b):
            pltpu.make_async_copy(
                bufs[t][b], outs[t].at[pl.ds(0, CH)], semo[t][b]).wait()

        for b in (0, 1):
            for t in range(nt):
                g(t, b, b)

        def step(cc, carry):
            for b in (0, 1):
                ch = 2 * cc + b
                for t in range(nt):
                    g_drain(t, b)
                    co(t, b, ch)
            for b in (0, 1):
                nch = jnp.minimum(2 * cc + 2 + b, NCHUNK - 1)
                for t in range(nt):
                    co_drain(t, b)
                    g(t, b, nch)
            return carry

        lax.fori_loop(0, NCHUNK // 2, step, 0)
        for b in (0, 1):
            for t in range(nt):
                g_drain(t, b)

    return body(*tables, *idx2ds)


def _sc_scatter_add(vals, idx2d):
    @functools.partial(
        pl.kernel,
        out_type=jax.ShapeDtypeStruct((NC * NPAD, F), _f32),
        mesh=_sc_mesh(),
        scratch_types=[
            pltpu.VMEM((NCHUNK, CH), jnp.int32),
            pltpu.VMEM((CH, F), _f32),
            pltpu.VMEM((CH, F), _f32),
            pltpu.VMEM_SHARED((NPAD, F), _f32),
            pltpu.SemaphoreType.DMA, pltpu.SemaphoreType.DMA,
            pltpu.SemaphoreType.DMA, pltpu.SemaphoreType.DMA,
        ],
    )
    def body(val_hbm, ii_hbm, out_hbm, idxv, b0, b1, acc,
             sl0, sl1, ss0, ss1):
        bufs = (b0, b1)
        seml = (sl0, sl1)
        sems = (ss0, ss1)
        c = lax.axis_index("c")
        s = lax.axis_index("s")
        w = s * NC + c
        ebase = w * EW
        rbase = w * NCHUNK
        pltpu.sync_copy(ii_hbm.at[pl.ds(rbase, NCHUNK)], idxv)

        def zrow(r, carry):
            for cc in range(F // 16):
                b0[r, pl.ds(cc * 16, 16)] = jnp.zeros((16,), _f32)
            return carry
        lax.fori_loop(0, CH, zrow, 0)

        def zacc(kk, carry):
            pltpu.sync_copy(b0, acc.at[pl.ds(s * ROWS_N + kk * CH, CH)])
            return carry
        lax.fori_loop(0, ROWS_N // CH, zacc, 0)
        plsc.subcore_barrier()

        def ld(b, ch):
            pltpu.async_copy(val_hbm.at[pl.ds(ebase + ch * CH, CH)],
                             bufs[b], seml[b])

        def ld_drain(b):
            pltpu.make_async_copy(val_hbm.at[pl.ds(0, CH)],
                                  bufs[b], seml[b]).wait()

        def sc(b, ch):
            pltpu.async_copy(bufs[b], acc.at[idxv.at[ch]], sems[b], add=True)

        def sc_drain(b):
            pltpu.make_async_copy(bufs[b], acc.at[pl.ds(0, CH)],
                                  sems[b]).wait()

        for b in (0, 1):
            ld(b, b)

        def step(cc, carry):
            for b in (0, 1):
                ch = 2 * cc + b
                ld_drain(b)
                sc(b, ch)
            for b in (0, 1):
                nch = jnp.minimum(2 * cc + 2 + b, NCHUNK - 1)
                sc_drain(b)
                ld(b, nch)
            return carry

        lax.fori_loop(0, NCHUNK // 2, step, 0)
        for b in (0, 1):
            ld_drain(b)
        plsc.subcore_barrier()

        pltpu.sync_copy(acc.at[pl.ds(s * ROWS_N, ROWS_N)],
                        out_hbm.at[pl.ds(c * NPAD + s * ROWS_N, ROWS_N)])

    return body(vals, idx2d)


def kernel(dr_vec, Z, idx, params):
    p = params
    pad_dr = jnp.broadcast_to(
        jnp.array([2.0 * CUTOFF, 0.0, 0.0], _f32), (EPAD - E, 3))
    drp = jnp.concatenate([dr_vec, pad_dr], axis=0)
    iidx = jnp.concatenate([idx[0], jnp.zeros((EPAD - E,), jnp.int32)])
    jidx = jnp.concatenate([idx[1], jnp.zeros((EPAD - E,), jnp.int32)])
    ii2d = iidx.reshape(NW * NCHUNK, CH)
    jj2d = jidx.reshape(NW * NCHUNK, CH)
    z2 = jnp.pad(Z, (0, NPAD - N)).reshape(NPAD, 1)

    embp = jnp.zeros((128, F), _f32).at[:119].set(p['embed'])
    wg16 = (jnp.zeros((F, F), _f32)
            .at[:, 0:3].set(p['Wqg']).at[:, 4:7].set(p['Wkg']))
    w1h = p['int_W1'][:F]
    w1d = jnp.zeros((8, F), _f32).at[0:3].set(p['int_W1'][F:])
    b1 = p['int_b1'].reshape(1, F)
    w2h = p['int_W2'][:, :F]
    b2h = p['int_b2'][:F].reshape(1, F)
    fb1 = p['filt_b1'].reshape(1, K)
    fb2 = p['filt_b2'].reshape(1, F)
    gb1 = p['gfilt_b1'].reshape(1, K)
    gb2 = p['gfilt_b2'].reshape(1, 3)
    rob1 = p['ro_b1'].reshape(1, F // 2)
    rob2 = p['ro_b2'].reshape(1, 1)
    sspack = (jnp.zeros((128, 8), _f32)
              .at[:119, 0:1].set(p['scale']).at[:119, 1:2].set(p['shift']))

    wkv = jnp.concatenate([p['Wk'], p['Wv']], axis=1)
    h, q_t, kv_t = _node_proj(z2, embp, p['Wq'], wkv)
    qg, kvg = _sc_gather([q_t, kv_t], [ii2d, jj2d], [F, 2 * F])
    msg = _edge_feat(drp, qg, kvg,
                     p['filt_W1'], fb1, p['filt_W2'], fb2)
    aggf = _sc_scatter_add(msg, ii2d)
    h2, g16 = _node_upd(h, aggf, p['Wo'], wg16)
    gi, gj = _sc_gather([g16, g16], [ii2d, jj2d], [F, F])
    ctr = _edge_geo(drp, gi, gj, p['gfilt_W1'], gb1, p['gfilt_W2'], gb2)
    chif = _sc_scatter_add(ctr, ii2d)
    out = _node_final(h2, chif, z2, w1h, w1d, b1, w2h, b2h,
                      p['ro_W1'], rob1, p['ro_W2'], rob2, sspack)
    return out[:N]


# speedup vs baseline: 1.0009x; 1.0009x over previous
import functools

import jax
import jax.numpy as jnp
import numpy as np
from jax import lax
from jax.experimental import pallas as pl
from jax.experimental.pallas import tpu as pltpu
from jax.experimental.pallas import tpu_sc as plsc

F = 128
K = 32
H = 4
DH = F // H
CUTOFF = 5.0
N = 10000
NPAD = 10240
E = 320000
EPAD = 327680
BN = 1024
GN = NPAD // BN
BE = 2048
GE = EPAD // BE
NC = 2
NS = 16
NW = NC * NS
CH = 128
EW = EPAD // NW
NCHUNK = EW // CH
ROWS_N = NPAD // NS

@functools.lru_cache(maxsize=1)
def _sc_mesh():
    return plsc.VectorSubcoreMesh(
        core_axis_name="c", subcore_axis_name="s",
        num_cores=NC, num_subcores=NS)

_f32 = jnp.float32


def _silu(x):
    return x / (1.0 + jnp.exp(-x))


def _node_proj_body(z_ref, emb_ref, wq_ref, wkv_ref,
                    h_ref, q_ref, kv_ref):
    z = z_ref[...]
    col = lax.broadcasted_iota(jnp.int32, (BN, 128), 1)
    oh = (col == z).astype(_f32)
    h = jnp.dot(oh, emb_ref[...], preferred_element_type=_f32)
    h_ref[...] = h
    q_ref[...] = jnp.dot(h, wq_ref[...], preferred_element_type=_f32)
    kv_ref[...] = jnp.dot(h, wkv_ref[...], preferred_element_type=_f32)


def _node_proj(z2, embp, wq, wkv):
    full = lambda a: pl.BlockSpec(a.shape, lambda i: (0,) * a.ndim)
    return pl.pallas_call(
        _node_proj_body,
        grid=(GN,),
        in_specs=[pl.BlockSpec((BN, 1), lambda i: (i, 0)),
                  full(embp), full(wq), full(wkv)],
        out_specs=[pl.BlockSpec((BN, F), lambda i: (i, 0)),
                   pl.BlockSpec((BN, F), lambda i: (i, 0)),
                   pl.BlockSpec((BN, 2 * F), lambda i: (i, 0))],
        out_shape=[jax.ShapeDtypeStruct((NPAD, F), _f32),
                   jax.ShapeDtypeStruct((NPAD, F), _f32),
                   jax.ShapeDtypeStruct((NPAD, 2 * F), _f32)],
    )(z2, embp, wq, wkv)


def _edge_geom(dr):
    r2 = jnp.sum(dr * dr, axis=1, keepdims=True)
    r = jnp.sqrt(r2 + 1e-12)
    rc = jnp.minimum(r, CUTOFF)
    cut = 0.5 * (jnp.cos(jnp.pi * rc / CUTOFF) + 1.0)
    cut = cut * (r < CUTOFF).astype(_f32)
    centers = (lax.broadcasted_iota(jnp.int32, (1, K), 1).astype(_f32)
               * (CUTOFF / (K - 1)))
    width = CUTOFF / K
    rbf = jnp.exp(-0.5 * ((r - centers) / width) ** 2)
    return r, cut, rbf


def _edge_feat_body(dr_ref, qg_ref, kvg_ref,
                    fw1_ref, fb1_ref, fw2_ref, fb2_ref, msg_ref):
    dr = dr_ref[...]
    _, cut, rbf = _edge_geom(dr)
    s1 = _silu(jnp.dot(rbf, fw1_ref[...], preferred_element_type=_f32)
               + fb1_ref[...])
    wf = jnp.dot(s1, fw2_ref[...], preferred_element_type=_f32) + fb2_ref[...]
    kvg = kvg_ref[...]
    t = qg_ref[...] * kvg[:, 0:F] * wf * (cut * (1.0 / np.sqrt(DH)))
    hr = lax.broadcasted_iota(jnp.int32, (F, F), 0) // DH
    hc = lax.broadcasted_iota(jnp.int32, (F, F), 1) // DH
    s_mat = (hr == hc).astype(_f32)
    alpha = jnp.dot(t, s_mat, preferred_element_type=_f32)
    msg_ref[...] = alpha * kvg[:, F:2 * F]


def _edge_feat(drp, qg, kvg, fw1, fb1, fw2, fb2):
    full = lambda a: pl.BlockSpec(a.shape, lambda i: (0,) * a.ndim)
    eb = pl.BlockSpec((BE, F), lambda i: (i, 0))
    return pl.pallas_call(
        _edge_feat_body,
        grid=(GE,),
        in_specs=[pl.BlockSpec((BE, 3), lambda i: (i, 0)), eb,
                  pl.BlockSpec((BE, 2 * F), lambda i: (i, 0)),
                  full(fw1), full(fb1), full(fw2), full(fb2)],
        out_specs=eb,
        out_shape=jax.ShapeDtypeStruct((EPAD, F), _f32),
    )(drp, qg, kvg, fw1, fb1, fw2, fb2)


def _node_upd_body(h_ref, a0_ref, a1_ref, wo_ref, wg_ref, h2_ref, g16_ref):
    agg = a0_ref[...] + a1_ref[...]
    h2 = h_ref[...] + jnp.dot(agg, wo_ref[...], preferred_element_type=_f32)
    h2_ref[...] = h2
    g16_ref[...] = jnp.dot(h2, wg_ref[...], preferred_element_type=_f32)


def _node_upd(h, aggf, wo, wg16):
    full = lambda a: pl.BlockSpec(a.shape, lambda i: (0,) * a.ndim)
    nb = pl.BlockSpec((BN, F), lambda i: (i, 0))
    return pl.pallas_call(
        _node_upd_body,
        grid=(GN,),
        in_specs=[nb, nb, pl.BlockSpec((BN, F), lambda i: (i + GN, 0)),
                  full(wo), full(wg16)],
        out_specs=[nb, nb],
        out_shape=[jax.ShapeDtypeStruct((NPAD, F), _f32),
                   jax.ShapeDtypeStruct((NPAD, F), _f32)],
    )(h, aggf, aggf, wo, wg16)


def _edge_geo_body(dr_ref, gi_ref, gj_ref,
                   gw1_ref, gb1_ref, gw2_ref, gb2_ref, ctr_ref):
    dr = dr_ref[...]
    r, cut, rbf = _edge_geom(dr)
    u = dr / r
    s1 = _silu(jnp.dot(rbf, gw1_ref[...], preferred_element_type=_f32)
               + gb1_ref[...])
    wg = jnp.dot(s1, gw2_ref[...], preferred_element_type=_f32) + gb2_ref[...]
    gi = gi_ref[...]
    gj = gj_ref[...]
    ag = gi[:, 0:3] * wg * gj[:, 4:7] * cut
    c16 = lax.broadcasted_iota(jnp.int32, (1, F), 1)
    agm = (ag[:, 0:1] * (c16 < 3).astype(_f32)
           + ag[:, 1:2] * ((c16 >= 3) & (c16 < 8)).astype(_f32)
           + ag[:, 2:3] * ((c16 >= 8) & (c16 < 15)).astype(_f32))
    x, y, z = u[:, 0:1], u[:, 1:2], u[:, 2:3]
    xx, yy, zz = x * x, y * y, z * z
    terms = [x, y, z,
             x * y, y * z, 3.0 * zz - 1.0, x * z, xx - yy,
             y * (3.0 * xx - yy), x * y * z, y * (5.0 * zz - 1.0),
             z * (5.0 * zz - 3.0), x * (5.0 * zz - 1.0),
             z * (xx - yy), x * (xx - 3.0 * yy)]
    sph = terms[0] * (c16 == 0).astype(_f32)
    for ti in range(1, 15):
        sph = sph + terms[ti] * (c16 == ti).astype(_f32)
    ctr_ref[...] = agm * sph


def _edge_geo(drp, gi, gj, gw1, gb1, gw2, gb2):
    full = lambda a: pl.BlockSpec(a.shape, lambda i: (0,) * a.ndim)
    eb = pl.BlockSpec((BE, F), lambda i: (i, 0))
    return pl.pallas_call(
        _edge_geo_body,
        grid=(GE,),
        in_specs=[pl.BlockSpec((BE, 3), lambda i: (i, 0)), eb, eb,
                  full(gw1), full(gb1), full(gw2), full(gb2)],
        out_specs=eb,
        out_shape=jax.ShapeDtypeStruct((EPAD, F), _f32),
    )(drp, gi, gj, gw1, gb1, gw2, gb2)


def _node_final_body(h2_ref, c0_ref, c1_ref, z_ref, w1h_ref, w1d_ref, b1_ref,
                     w2h_ref, b2h_ref, row1_ref, rob1_ref, row2_ref, rob2_ref,
                     ss_ref, out_ref):
    chi = c0_ref[...] + c1_ref[...]
    d1 = jnp.sum(chi[:, 0:3] ** 2, axis=1, keepdims=True)
    d2 = jnp.sum(chi[:, 3:8] ** 2, axis=1, keepdims=True)
    d3 = jnp.sum(chi[:, 8:15] ** 2, axis=1, keepdims=True)
    h2 = h2_ref[...]
    w1d = w1d_ref[...]
    pre = (jnp.dot(h2, w1h_ref[...], preferred_element_type=_f32)
           + d1 * w1d[0:1, :] + d2 * w1d[1:2, :] + d3 * w1d[2:3, :]
           + b1_ref[...])
    s = _silu(pre)
    h3 = h2 + jnp.dot(s, w2h_ref[...], preferred_element_type=_f32) + b2h_ref[...]
    t = _silu(jnp.dot(h3, row1_ref[...], preferred_element_type=_f32)
              + rob1_ref[...])
    e = jnp.dot(t, row2_ref[...], preferred_element_type=_f32) + rob2_ref[...]
    z = z_ref[...]
    nm = (z > 0).astype(_f32)
    oh = (lax.broadcasted_iota(jnp.int32, (BN, 128), 1) == z).astype(_f32)
    ss = jnp.dot(oh, ss_ref[...], preferred_element_type=_f32)
    out_ref[...] = ss[:, 0:1] * (e * nm) + ss[:, 1:2]


def _node_final(h2, chif, z2, w1h, w1d, b1, w2h, b2h,
                row1, rob1, row2, rob2, sspack):
    full = lambda a: pl.BlockSpec(a.shape, lambda i: (0,) * a.ndim)
    nb = pl.BlockSpec((BN, F), lambda i: (i, 0))
    return pl.pallas_call(
        _node_final_body,
        grid=(GN,),
        in_specs=[nb,
                  nb, pl.BlockSpec((BN, F), lambda i: (i + GN, 0)),
                  pl.BlockSpec((BN, 1), lambda i: (i, 0)),
                  full(w1h), full(w1d), full(b1), full(w2h), full(b2h),
                  full(row1), full(rob1), full(row2), full(rob2),
                  full(sspack)],
        out_specs=pl.BlockSpec((BN, 1), lambda i: (i, 0)),
        out_shape=jax.ShapeDtypeStruct((NPAD, 1), _f32),
    )(h2, chif, chif, z2, w1h, w1d, b1, w2h, b2h,
      row1, rob1, row2, rob2, sspack)


def _sc_gather(tables, idx2ds, widths, name):
    nt = len(tables)
    scratch = ([pltpu.VMEM((NCHUNK, CH), jnp.int32) for _ in range(nt)]
               + [pltpu.VMEM((CH, w), _f32) for w in widths for _ in (0, 1)]
               + [pltpu.SemaphoreType.DMA] * (4 * nt))

    @functools.partial(
        pl.kernel,
        out_type=[jax.ShapeDtypeStruct((EPAD, w), _f32) for w in widths],
        mesh=_sc_mesh(),
        scratch_types=scratch,
        name=name,
    )
    def body(*refs):
        tabs = refs[:nt]
        idxh = refs[nt:2 * nt]
        outs = refs[2 * nt:3 * nt]
        idxv = refs[3 * nt:4 * nt]
        bufs = [refs[4 * nt + 2 * t:4 * nt + 2 * t + 2] for t in range(nt)]
        semg = [refs[6 * nt + 2 * t:6 * nt + 2 * t + 2] for t in range(nt)]
        semo = [refs[8 * nt + 2 * t:8 * nt + 2 * t + 2] for t in range(nt)]
        w = lax.axis_index("s") * NC + lax.axis_index("c")
        ebase = w * EW
        rbase = w * NCHUNK
        for t in range(nt):
            pltpu.sync_copy(idxh[t].at[pl.ds(rbase, NCHUNK)], idxv[t])

        def g(t, b, ch):
            pltpu.async_copy(tabs[t].at[idxv[t].at[ch]], bufs[t][b], semg[t][b])

        def g_drain(t, b):
            pltpu.make_async_copy(
                tabs[t].at[pl.ds(0, CH)], bufs[t][b], semg[t][b]).wait()

        def co(t, b, ch):
            pltpu.async_copy(bufs[t][b],
                             outs[t].at[pl.ds(ebase + ch * CH, CH)],
                             semo[t][b])

        def co_drain(t, b):
            pltpu.make_async_copy(
                bufs[t][b], outs[t].at[pl.ds(0, CH)], semo[t][b]).wait()

        for b in (0, 1):
            for t in range(nt):
                g(t, b, b)

        def step(cc, carry):
            for b in (0, 1):
                ch = 2 * cc + b
                for t in range(nt):
                    g_drain(t, b)
                    co(t, b, ch)
            for b in (0, 1):
                nch = jnp.minimum(2 * cc + 2 + b, NCHUNK - 1)
                for t in range(nt):
                    co_drain(t, b)
                    g(t, b, nch)
            return carry

        lax.fori_loop(0, NCHUNK // 2, step, 0)
        for b in (0, 1):
            for t in range(nt):
                g_drain(t, b)

    return body(*tables, *idx2ds)


def _sc_scatter_add(vals, idx2d, name):
    @functools.partial(
        pl.kernel,
        out_type=jax.ShapeDtypeStruct((NC * NPAD, F), _f32),
        mesh=_sc_mesh(),
        name=name,
        scratch_types=[
            pltpu.VMEM((NCHUNK, CH), jnp.int32),
            pltpu.VMEM((CH, F), _f32),
            pltpu.VMEM((CH, F), _f32),
            pltpu.VMEM_SHARED((NPAD, F), _f32),
            pltpu.SemaphoreType.DMA, pltpu.SemaphoreType.DMA,
            pltpu.SemaphoreType.DMA, pltpu.SemaphoreType.DMA,
        ],
    )
    def body(val_hbm, ii_hbm, out_hbm, idxv, b0, b1, acc,
             sl0, sl1, ss0, ss1):
        bufs = (b0, b1)
        seml = (sl0, sl1)
        sems = (ss0, ss1)
        c = lax.axis_index("c")
        s = lax.axis_index("s")
        w = s * NC + c
        ebase = w * EW
        rbase = w * NCHUNK
        pltpu.sync_copy(ii_hbm.at[pl.ds(rbase, NCHUNK)], idxv)

        def zrow(r, carry):
            for cc in range(F // 16):
                b0[r, pl.ds(cc * 16, 16)] = jnp.zeros((16,), _f32)
            return carry
        lax.fori_loop(0, CH, zrow, 0)

        def zacc(kk, carry):
            pltpu.sync_copy(b0, acc.at[pl.ds(s * ROWS_N + kk * CH, CH)])
            return carry
        lax.fori_loop(0, ROWS_N // CH, zacc, 0)
        plsc.subcore_barrier()

        def ld(b, ch):
            pltpu.async_copy(val_hbm.at[pl.ds(ebase + ch * CH, CH)],
                             bufs[b], seml[b])

        def ld_drain(b):
            pltpu.make_async_copy(val_hbm.at[pl.ds(0, CH)],
                                  bufs[b], seml[b]).wait()

        def sc(b, ch):
            pltpu.async_copy(bufs[b], acc.at[idxv.at[ch]], sems[b], add=True)

        def sc_drain(b):
            pltpu.make_async_copy(bufs[b], acc.at[pl.ds(0, CH)],
                                  sems[b]).wait()

        for b in (0, 1):
            ld(b, b)

        def step(cc, carry):
            for b in (0, 1):
                ch = 2 * cc + b
                ld_drain(b)
                sc(b, ch)
            for b in (0, 1):
                nch = jnp.minimum(2 * cc + 2 + b, NCHUNK - 1)
                sc_drain(b)
                ld(b, nch)
            return carry

        lax.fori_loop(0, NCHUNK // 2, step, 0)
        for b in (0, 1):
            ld_drain(b)
        plsc.subcore_barrier()

        pltpu.sync_copy(acc.at[pl.ds(s * ROWS_N, ROWS_N)],
                        out_hbm.at[pl.ds(c * NPAD + s * ROWS_N, ROWS_N)])

    return body(vals, idx2d)


def kernel(dr_vec, Z, idx, params):
    p = params
    pad_dr = jnp.broadcast_to(
        jnp.array([2.0 * CUTOFF, 0.0, 0.0], _f32), (EPAD - E, 3))
    drp = jnp.concatenate([dr_vec, pad_dr], axis=0)
    iidx = jnp.concatenate([idx[0], jnp.zeros((EPAD - E,), jnp.int32)])
    jidx = jnp.concatenate([idx[1], jnp.zeros((EPAD - E,), jnp.int32)])
    ii2d = iidx.reshape(NW * NCHUNK, CH)
    jj2d = jidx.reshape(NW * NCHUNK, CH)
    z2 = jnp.pad(Z, (0, NPAD - N)).reshape(NPAD, 1)

    embp = jnp.zeros((128, F), _f32).at[:119].set(p['embed'])
    wg16 = (jnp.zeros((F, F), _f32)
            .at[:, 0:3].set(p['Wqg']).at[:, 4:7].set(p['Wkg']))
    w1h = p['int_W1'][:F]
    w1d = jnp.zeros((8, F), _f32).at[0:3].set(p['int_W1'][F:])
    b1 = p['int_b1'].reshape(1, F)
    w2h = p['int_W2'][:, :F]
    b2h = p['int_b2'][:F].reshape(1, F)
    fb1 = p['filt_b1'].reshape(1, K)
    fb2 = p['filt_b2'].reshape(1, F)
    gb1 = p['gfilt_b1'].reshape(1, K)
    gb2 = p['gfilt_b2'].reshape(1, 3)
    rob1 = p['ro_b1'].reshape(1, F // 2)
    rob2 = p['ro_b2'].reshape(1, 1)
    sspack = (jnp.zeros((128, 8), _f32)
              .at[:119, 0:1].set(p['scale']).at[:119, 1:2].set(p['shift']))

    wkv = jnp.concatenate([p['Wk'], p['Wv']], axis=1)
    h, q_t, kv_t = _node_proj(z2, embp, p['Wq'], wkv)
    qg, kvg = _sc_gather([q_t, kv_t], [ii2d, jj2d], [F, 2 * F], 'sc_gather_qkv')
    msg = _edge_feat(drp, qg, kvg,
                     p['filt_W1'], fb1, p['filt_W2'], fb2)
    aggf = _sc_scatter_add(msg, ii2d, 'sc_scatter_msg')
    h2, g16 = _node_upd(h, aggf, p['Wo'], wg16)
    gi, gj = _sc_gather([g16, g16], [ii2d, jj2d], [F, F], 'sc_gather_geo')
    ctr = _edge_geo(drp, gi, gj, p['gfilt_W1'], gb1, p['gfilt_W2'], gb2)
    chif = _sc_scatter_add(ctr, ii2d, 'sc_scatter_chi')
    out = _node_final(h2, chif, z2, w1h, w1d, b1, w2h, b2h,
                      p['ro_W1'], rob1, p['ro_W2'], rob2, sspack)
    return out[:N]


# speedup vs baseline: 1.4480x; 1.4467x over previous
import functools

import jax
import jax.numpy as jnp
import numpy as np
from jax import lax
from jax.experimental import pallas as pl
from jax.experimental.pallas import tpu as pltpu
from jax.experimental.pallas import tpu_sc as plsc

F = 128
K = 32
H = 4
DH = F // H
CUTOFF = 5.0
N = 10000
NPAD = 10240
E = 320000
EPAD = 327680
BN = 1024
GN = NPAD // BN
BE = 2048
GE = EPAD // BE
NC = 2
NS = 16
NW = NC * NS
CH = 128
EW = EPAD // NW
NCHUNK = EW // CH
ROWS_N = NPAD // NS

@functools.lru_cache(maxsize=1)
def _sc_mesh():
    return plsc.VectorSubcoreMesh(
        core_axis_name="c", subcore_axis_name="s",
        num_cores=NC, num_subcores=NS)

_f32 = jnp.float32


def _silu(x):
    return x / (1.0 + jnp.exp(-x))


def _node_proj_body(z_ref, emb_ref, wq_ref, wk_ref, wv_ref,
                    h_ref, q_ref, kv_ref):
    z = z_ref[...]
    col = lax.broadcasted_iota(jnp.int32, (BN, 128), 1)
    oh = (col == z).astype(_f32)
    h = jnp.dot(oh, emb_ref[...], preferred_element_type=_f32)
    h_ref[...] = h
    q_ref[...] = jnp.dot(h, wq_ref[...], preferred_element_type=_f32)
    k = jnp.dot(h, wk_ref[...], preferred_element_type=_f32)
    v = jnp.dot(h, wv_ref[...], preferred_element_type=_f32)
    ku = lax.bitcast_convert_type(k, jnp.uint32)
    kr = (ku + jnp.uint32(0x7FFF) + ((ku >> 16) & jnp.uint32(1))) >> 16
    vu = lax.bitcast_convert_type(v, jnp.uint32)
    vr = (vu + jnp.uint32(0x7FFF) + ((vu >> 16) & jnp.uint32(1))) \
        & jnp.uint32(0xFFFF0000)
    kv_ref[...] = vr | kr


def _node_proj(z2, embp, wq, wk, wv):
    full = lambda a: pl.BlockSpec(a.shape, lambda i: (0,) * a.ndim)
    return pl.pallas_call(
        _node_proj_body,
        grid=(GN,),
        in_specs=[pl.BlockSpec((BN, 1), lambda i: (i, 0)),
                  full(embp), full(wq), full(wk), full(wv)],
        out_specs=[pl.BlockSpec((BN, F), lambda i: (i, 0)),
                   pl.BlockSpec((BN, F), lambda i: (i, 0)),
                   pl.BlockSpec((BN, F), lambda i: (i, 0))],
        out_shape=[jax.ShapeDtypeStruct((NPAD, F), _f32),
                   jax.ShapeDtypeStruct((NPAD, F), _f32),
                   jax.ShapeDtypeStruct((NPAD, F), jnp.uint32)],
    )(z2, embp, wq, wk, wv)


def _edge_geom(dr):
    r2 = jnp.sum(dr * dr, axis=1, keepdims=True)
    r = jnp.sqrt(r2 + 1e-12)
    rc = jnp.minimum(r, CUTOFF)
    cut = 0.5 * (jnp.cos(jnp.pi * rc / CUTOFF) + 1.0)
    cut = cut * (r < CUTOFF).astype(_f32)
    centers = (lax.broadcasted_iota(jnp.int32, (1, K), 1).astype(_f32)
               * (CUTOFF / (K - 1)))
    width = CUTOFF / K
    rbf = jnp.exp(-0.5 * ((r - centers) / width) ** 2)
    return r, cut, rbf


def _edge_feat_body(dr_ref, qg_ref, kvg_ref,
                    fw1_ref, fb1_ref, fw2_ref, fb2_ref, msg_ref):
    dr = dr_ref[...]
    _, cut, rbf = _edge_geom(dr)
    s1 = _silu(jnp.dot(rbf, fw1_ref[...], preferred_element_type=_f32)
               + fb1_ref[...])
    wf = jnp.dot(s1, fw2_ref[...], preferred_element_type=_f32) + fb2_ref[...]
    kvg = kvg_ref[...]
    kg = lax.bitcast_convert_type(kvg << 16, _f32)
    vg = lax.bitcast_convert_type(kvg & jnp.uint32(0xFFFF0000), _f32)
    t = qg_ref[...] * kg * wf * (cut * (1.0 / np.sqrt(DH)))
    hr = lax.broadcasted_iota(jnp.int32, (F, F), 0) // DH
    hc = lax.broadcasted_iota(jnp.int32, (F, F), 1) // DH
    s_mat = (hr == hc).astype(_f32)
    alpha = jnp.dot(t, s_mat, preferred_element_type=_f32)
    msg_ref[...] = alpha * vg


def _edge_feat(drp, qg, kvg, fw1, fb1, fw2, fb2):
    full = lambda a: pl.BlockSpec(a.shape, lambda i: (0,) * a.ndim)
    eb = pl.BlockSpec((BE, F), lambda i: (i, 0))
    return pl.pallas_call(
        _edge_feat_body,
        grid=(GE,),
        in_specs=[pl.BlockSpec((BE, 3), lambda i: (i, 0)), eb, eb,
                  full(fw1), full(fb1), full(fw2), full(fb2)],
        out_specs=eb,
        out_shape=jax.ShapeDtypeStruct((EPAD, F), _f32),
    )(drp, qg, kvg, fw1, fb1, fw2, fb2)


def _node_upd_body(h_ref, a0_ref, a1_ref, wo_ref, wg_ref, h2_ref, g16_ref):
    agg = a0_ref[...] + a1_ref[...]
    h2 = h_ref[...] + jnp.dot(agg, wo_ref[...], preferred_element_type=_f32)
    h2_ref[...] = h2
    g16_ref[...] = jnp.dot(h2, wg_ref[...], preferred_element_type=_f32)


def _node_upd(h, aggf, wo, wg16):
    full = lambda a: pl.BlockSpec(a.shape, lambda i: (0,) * a.ndim)
    nb = pl.BlockSpec((BN, F), lambda i: (i, 0))
    return pl.pallas_call(
        _node_upd_body,
        grid=(GN,),
        in_specs=[nb, nb, pl.BlockSpec((BN, F), lambda i: (i + GN, 0)),
                  full(wo), full(wg16)],
        out_specs=[nb, nb],
        out_shape=[jax.ShapeDtypeStruct((NPAD, F), _f32),
                   jax.ShapeDtypeStruct((NPAD, F), _f32)],
    )(h, aggf, aggf, wo, wg16)


def _edge_geo_body(dr_ref, gj_ref,
                   gw1_ref, gb1_ref, gw2_ref, gb2_ref, ctr_ref):
    dr = dr_ref[...]
    r, cut, rbf = _edge_geom(dr)
    u = dr / r
    s1 = _silu(jnp.dot(rbf, gw1_ref[...], preferred_element_type=_f32)
               + gb1_ref[...])
    wg = jnp.dot(s1, gw2_ref[...], preferred_element_type=_f32) + gb2_ref[...]
    gj = gj_ref[...]
    ag = wg * gj[:, 4:7] * cut
    c16 = lax.broadcasted_iota(jnp.int32, (1, F), 1)
    agm = (ag[:, 0:1] * (c16 < 3).astype(_f32)
           + ag[:, 1:2] * ((c16 >= 3) & (c16 < 8)).astype(_f32)
           + ag[:, 2:3] * ((c16 >= 8) & (c16 < 15)).astype(_f32))
    x, y, z = u[:, 0:1], u[:, 1:2], u[:, 2:3]
    xx, yy, zz = x * x, y * y, z * z
    terms = [x, y, z,
             x * y, y * z, 3.0 * zz - 1.0, x * z, xx - yy,
             y * (3.0 * xx - yy), x * y * z, y * (5.0 * zz - 1.0),
             z * (5.0 * zz - 3.0), x * (5.0 * zz - 1.0),
             z * (xx - yy), x * (xx - 3.0 * yy)]
    sph = terms[0] * (c16 == 0).astype(_f32)
    for ti in range(1, 15):
        sph = sph + terms[ti] * (c16 == ti).astype(_f32)
    ctr_ref[...] = agm * sph


def _edge_geo(drp, gj, gw1, gb1, gw2, gb2):
    full = lambda a: pl.BlockSpec(a.shape, lambda i: (0,) * a.ndim)
    eb = pl.BlockSpec((BE, F), lambda i: (i, 0))
    return pl.pallas_call(
        _edge_geo_body,
        grid=(GE,),
        in_specs=[pl.BlockSpec((BE, 3), lambda i: (i, 0)), eb,
                  full(gw1), full(gb1), full(gw2), full(gb2)],
        out_specs=eb,
        out_shape=jax.ShapeDtypeStruct((EPAD, F), _f32),
    )(drp, gj, gw1, gb1, gw2, gb2)


def _node_final_body(h2_ref, c0_ref, c1_ref, g16_ref, z_ref,
                     w1h_ref, w1d_ref, b1_ref,
                     w2h_ref, b2h_ref, row1_ref, rob1_ref, row2_ref, rob2_ref,
                     ss_ref, out_ref):
    g16 = g16_ref[...]
    c16 = lax.broadcasted_iota(jnp.int32, (1, F), 1)
    qgrep = (g16[:, 0:1] * (c16 < 3).astype(_f32)
             + g16[:, 1:2] * ((c16 >= 3) & (c16 < 8)).astype(_f32)
             + g16[:, 2:3] * ((c16 >= 8) & (c16 < 15)).astype(_f32))
    chi = (c0_ref[...] + c1_ref[...]) * qgrep
    d1 = jnp.sum(chi[:, 0:3] ** 2, axis=1, keepdims=True)
    d2 = jnp.sum(chi[:, 3:8] ** 2, axis=1, keepdims=True)
    d3 = jnp.sum(chi[:, 8:15] ** 2, axis=1, keepdims=True)
    h2 = h2_ref[...]
    w1d = w1d_ref[...]
    pre = (jnp.dot(h2, w1h_ref[...], preferred_element_type=_f32)
           + d1 * w1d[0:1, :] + d2 * w1d[1:2, :] + d3 * w1d[2:3, :]
           + b1_ref[...])
    s = _silu(pre)
    h3 = h2 + jnp.dot(s, w2h_ref[...], preferred_element_type=_f32) + b2h_ref[...]
    t = _silu(jnp.dot(h3, row1_ref[...], preferred_element_type=_f32)
              + rob1_ref[...])
    e = jnp.dot(t, row2_ref[...], preferred_element_type=_f32) + rob2_ref[...]
    z = z_ref[...]
    nm = (z > 0).astype(_f32)
    oh = (lax.broadcasted_iota(jnp.int32, (BN, 128), 1) == z).astype(_f32)
    ss = jnp.dot(oh, ss_ref[...], preferred_element_type=_f32)
    out_ref[...] = ss[:, 0:1] * (e * nm) + ss[:, 1:2]


def _node_final(h2, chif, g16, z2, w1h, w1d, b1, w2h, b2h,
                row1, rob1, row2, rob2, sspack):
    full = lambda a: pl.BlockSpec(a.shape, lambda i: (0,) * a.ndim)
    nb = pl.BlockSpec((BN, F), lambda i: (i, 0))
    return pl.pallas_call(
        _node_final_body,
        grid=(GN,),
        in_specs=[nb,
                  nb, pl.BlockSpec((BN, F), lambda i: (i + GN, 0)),
                  nb,
                  pl.BlockSpec((BN, 1), lambda i: (i, 0)),
                  full(w1h), full(w1d), full(b1), full(w2h), full(b2h),
                  full(row1), full(rob1), full(row2), full(rob2),
                  full(sspack)],
        out_specs=pl.BlockSpec((BN, 1), lambda i: (i, 0)),
        out_shape=jax.ShapeDtypeStruct((NPAD, 1), _f32),
    )(h2, chif, chif, g16, z2, w1h, w1d, b1, w2h, b2h,
      row1, rob1, row2, rob2, sspack)


def _sc_gather(tables, idx2ds, name, spmem):
    nt = len(tables)
    rests = [t.shape[1:] for t in tables]
    dts = [t.dtype for t in tables]
    scratch = ([pltpu.VMEM((NCHUNK, CH), jnp.int32) for _ in range(nt)]
               + [pltpu.VMEM((CH,) + rests[t], dts[t])
                  for t in range(nt) for _ in (0, 1)]
               + [pltpu.SemaphoreType.DMA] * (4 * nt)
               + [pltpu.VMEM_SHARED((NPAD,) + rests[t], dts[t])
                  for t in range(nt) if spmem[t]])

    @functools.partial(
        pl.kernel,
        out_type=[jax.ShapeDtypeStruct((EPAD,) + rests[t], dts[t])
                  for t in range(nt)],
        mesh=_sc_mesh(),
        scratch_types=scratch,
        name=name,
    )
    def body(*refs):
        tabs = refs[:nt]
        idxh = refs[nt:2 * nt]
        outs = refs[2 * nt:3 * nt]
        idxv = refs[3 * nt:4 * nt]
        bufs = [refs[4 * nt + 2 * t:4 * nt + 2 * t + 2] for t in range(nt)]
        semg = [refs[6 * nt + 2 * t:6 * nt + 2 * t + 2] for t in range(nt)]
        semo = [refs[8 * nt + 2 * t:8 * nt + 2 * t + 2] for t in range(nt)]
        spms = list(refs[10 * nt:])
        srcs = [spms.pop(0) if spmem[t] else tabs[t] for t in range(nt)]
        s = lax.axis_index("s")
        w = s * NC + lax.axis_index("c")
        ebase = w * EW
        rbase = w * NCHUNK
        for t in range(nt):
            pltpu.sync_copy(idxh[t].at[pl.ds(rbase, NCHUNK)], idxv[t])
        staged = False
        for t in range(nt):
            if spmem[t]:
                pltpu.sync_copy(tabs[t].at[pl.ds(s * ROWS_N, ROWS_N)],
                                srcs[t].at[pl.ds(s * ROWS_N, ROWS_N)])
                staged = True
        if staged:
            plsc.subcore_barrier()

        def g(t, b, ch):
            pltpu.async_copy(srcs[t].at[idxv[t].at[ch]], bufs[t][b], semg[t][b])

        def g_drain(t, b):
            pltpu.make_async_copy(
                srcs[t].at[pl.ds(0, CH)], bufs[t][b], semg[t][b]).wait()

        def co(t, b, ch):
            pltpu.async_copy(bufs[t][b],
                             outs[t].at[pl.ds(ebase + ch * CH, CH)],
                             semo[t][b])

        def co_drain(t, b):
            pltpu.make_async_copy(
                bufs[t][b], outs[t].at[pl.ds(0, CH)], semo[t][b]).wait()

        for b in (0, 1):
            for t in range(nt):
                g(t, b, b)

        def step(cc, carry):
            for b in (0, 1):
                ch = 2 * cc + b
                for t in range(nt):
                    g_drain(t, b)
                    co(t, b, ch)
            for b in (0, 1):
                nch = jnp.minimum(2 * cc + 2 + b, NCHUNK - 1)
                for t in range(nt):
                    co_drain(t, b)
                    g(t, b, nch)
            return carry

        lax.fori_loop(0, NCHUNK // 2, step, 0)
        for b in (0, 1):
            for t in range(nt):
                g_drain(t, b)

    return body(*tables, *idx2ds)


def _sc_scatter_add(vals, idx2d, name):
    @functools.partial(
        pl.kernel,
        out_type=jax.ShapeDtypeStruct((NC * NPAD, F), _f32),
        mesh=_sc_mesh(),
        name=name,
        scratch_types=[
            pltpu.VMEM((NCHUNK, CH), jnp.int32),
            pltpu.VMEM((CH, F), _f32),
            pltpu.VMEM((CH, F), _f32),
            pltpu.VMEM_SHARED((NPAD, F), _f32),
            pltpu.SemaphoreType.DMA, pltpu.SemaphoreType.DMA,
            pltpu.SemaphoreType.DMA, pltpu.SemaphoreType.DMA,
        ],
    )
    def body(val_hbm, ii_hbm, out_hbm, idxv, b0, b1, acc,
             sl0, sl1, ss0, ss1):
        bufs = (b0, b1)
        seml = (sl0, sl1)
        sems = (ss0, ss1)
        c = lax.axis_index("c")
        s = lax.axis_index("s")
        w = s * NC + c
        ebase = w * EW
        rbase = w * NCHUNK
        pltpu.sync_copy(ii_hbm.at[pl.ds(rbase, NCHUNK)], idxv)

        def zrow(r, carry):
            for cc in range(F // 16):
                b0[r, pl.ds(cc * 16, 16)] = jnp.zeros((16,), _f32)
            return carry
        lax.fori_loop(0, CH, zrow, 0)

        def zacc(kk, carry):
            pltpu.sync_copy(b0, acc.at[pl.ds(s * ROWS_N + kk * CH, CH)])
            return carry
        lax.fori_loop(0, ROWS_N // CH, zacc, 0)
        plsc.subcore_barrier()

        def ld(b, ch):
            pltpu.async_copy(val_hbm.at[pl.ds(ebase + ch * CH, CH)],
                             bufs[b], seml[b])

        def ld_drain(b):
            pltpu.make_async_copy(val_hbm.at[pl.ds(0, CH)],
                                  bufs[b], seml[b]).wait()

        def sc(b, ch):
            pltpu.async_copy(bufs[b], acc.at[idxv.at[ch]], sems[b], add=True)

        def sc_drain(b):
            pltpu.make_async_copy(bufs[b], acc.at[pl.ds(0, CH)],
                                  sems[b]).wait()

        for b in (0, 1):
            ld(b, b)

        def step(cc, carry):
            for b in (0, 1):
                ch = 2 * cc + b
                ld_drain(b)
                sc(b, ch)
            for b in (0, 1):
                nch = jnp.minimum(2 * cc + 2 + b, NCHUNK - 1)
                sc_drain(b)
                ld(b, nch)
            return carry

        lax.fori_loop(0, NCHUNK // 2, step, 0)
        for b in (0, 1):
            ld_drain(b)
        plsc.subcore_barrier()

        pltpu.sync_copy(acc.at[pl.ds(s * ROWS_N, ROWS_N)],
                        out_hbm.at[pl.ds(c * NPAD + s * ROWS_N, ROWS_N)])

    return body(vals, idx2d)


def kernel(dr_vec, Z, idx, params):
    p = params
    pad_dr = jnp.broadcast_to(
        jnp.array([2.0 * CUTOFF, 0.0, 0.0], _f32), (EPAD - E, 3))
    drp = jnp.concatenate([dr_vec, pad_dr], axis=0)
    iidx = jnp.concatenate([idx[0], jnp.zeros((EPAD - E,), jnp.int32)])
    jidx = jnp.concatenate([idx[1], jnp.zeros((EPAD - E,), jnp.int32)])
    ii2d = iidx.reshape(NW * NCHUNK, CH)
    jj2d = jidx.reshape(NW * NCHUNK, CH)
    z2 = jnp.pad(Z, (0, NPAD - N)).reshape(NPAD, 1)

    embp = jnp.zeros((128, F), _f32).at[:119].set(p['embed'])
    wg16 = (jnp.zeros((F, F), _f32)
            .at[:, 0:3].set(p['Wqg']).at[:, 4:7].set(p['Wkg']))
    w1h = p['int_W1'][:F]
    w1d = jnp.zeros((8, F), _f32).at[0:3].set(p['int_W1'][F:])
    b1 = p['int_b1'].reshape(1, F)
    w2h = p['int_W2'][:, :F]
    b2h = p['int_b2'][:F].reshape(1, F)
    fb1 = p['filt_b1'].reshape(1, K)
    fb2 = p['filt_b2'].reshape(1, F)
    gb1 = p['gfilt_b1'].reshape(1, K)
    gb2 = p['gfilt_b2'].reshape(1, 3)
    rob1 = p['ro_b1'].reshape(1, F // 2)
    rob2 = p['ro_b2'].reshape(1, 1)
    sspack = (jnp.zeros((128, 8), _f32)
              .at[:119, 0:1].set(p['scale']).at[:119, 1:2].set(p['shift']))

    h, q_t, kv_t = _node_proj(z2, embp, p['Wq'], p['Wk'], p['Wv'])
    (qg,) = _sc_gather([q_t], [ii2d], 'sc_gather_q', [True])
    (kvg,) = _sc_gather([kv_t], [jj2d], 'sc_gather_kv', [True])
    msg = _edge_feat(drp, qg, kvg,
                     p['filt_W1'], fb1, p['filt_W2'], fb2)
    aggf = _sc_scatter_add(msg, ii2d, 'sc_scatter_msg')
    h2, g16 = _node_upd(h, aggf, p['Wo'], wg16)
    (gj,) = _sc_gather([g16], [jj2d], 'sc_gather_geo', [True])
    ctr = _edge_geo(drp, gj, p['gfilt_W1'], gb1, p['gfilt_W2'], gb2)
    chif = _sc_scatter_add(ctr, ii2d, 'sc_scatter_chi')
    out = _node_final(h2, chif, g16, z2, w1h, w1d, b1, w2h, b2h,
                      p['ro_W1'], rob1, p['ro_W2'], rob2, sspack)
    return out[:N]
